# Initial kernel scaffold; baseline (speedup 1.0000x reference)
import jax
import jax.numpy as jnp
from jax.experimental import pallas as pl


def kernel(node_feats, edge_index, edge_attr, params):
    raise NotImplementedError("write your pallas kernel here")


# speedup vs baseline: 2.3962x; 2.3962x over previous
import functools

import jax
import jax.numpy as jnp
from jax import lax
from jax.experimental import pallas as pl
from jax.experimental.pallas import tpu as pltpu
from jax.experimental.pallas import tpu_sc as plsc

F32 = jnp.float32
DEPTH = 3
NC = 2
NS = 16
NW = NC * NS
NPAD = 10240


def _dot(x, w):
    return jnp.dot(x, w, preferred_element_type=F32)


def _mlp_pack(p, H=64):
    lins, bns = p["lins"], p["bns"]
    Ws = [W.T for (W, b) in lins]
    bs = [b.reshape(1, -1) for (W, b) in lins]
    bn = []
    for g, bb, m, v in bns:
        d = jnp.sqrt(v + 1e-5)
        bn += [m.reshape(1, H), d.reshape(1, H),
               g.reshape(1, H), bb.reshape(1, H)]
    return Ws, bs, bn


def _mlp_seq(x, mats, biases, bns):
    x = _dot(x, mats[0]) + biases[0]
    x = _dot(x, mats[1]) + biases[1]
    x = jnp.maximum((x - bns[0]) / bns[1] * bns[2] + bns[3], 0.0)
    x = _dot(x, mats[2]) + biases[2]
    x = _dot(x, mats[3]) + biases[3]
    x = jnp.maximum((x - bns[4]) / bns[5] * bns[6] + bns[7], 0.0)
    return _dot(x, mats[4]) + biases[4]


def _unstack(ref, n):
    return [ref[i] for i in range(n)]


def _node_init_body(x_ref, w0_ref, wr_ref, b_ref, bn_ref, wio_ref,
                    nf_ref, ti_ref, tj_ref):
    mats = [w0_ref[...]] + _unstack(wr_ref, 4)
    nf = _mlp_seq(x_ref[...], mats, _unstack(b_ref, 5), _unstack(bn_ref, 8))
    nf_ref[...] = nf
    tij = _dot(nf, wio_ref[...])
    z = jnp.zeros_like(tij[:, :64])
    ti_ref[...] = jnp.concatenate([tij[:, :64], z], axis=1)
    tj_ref[...] = jnp.concatenate([tij[:, 64:], z], axis=1)


def _node_init_call(x, w0, wr, b, bn, wio, N, H):
    t_sh = jax.ShapeDtypeStruct((N, 2 * H), F32)
    return pl.pallas_call(
        _node_init_body,
        out_shape=[jax.ShapeDtypeStruct((N, H), F32), t_sh, t_sh],
    )(x, w0, wr, b, bn, wio)


def _edge_init_body(x_ref, w0_ref, wr_ref, b_ref, bn_ref, ea_ref):
    mats = [w0_ref[...]] + _unstack(wr_ref, 4)
    ea_ref[...] = _mlp_seq(x_ref[...], mats, _unstack(b_ref, 5),
                           _unstack(bn_ref, 8))


def _edge_init_call(x, w0, wr, b, bn, E, H, BE):
    d_in = x.shape[1]
    return pl.pallas_call(
        _edge_init_body,
        grid=(E // BE,),
        in_specs=[
            pl.BlockSpec((BE, d_in), lambda i: (i, 0)),
            pl.BlockSpec(w0.shape, lambda i: (0, 0)),
            pl.BlockSpec(wr.shape, lambda i: (0, 0, 0)),
            pl.BlockSpec(b.shape, lambda i: (0, 0, 0)),
            pl.BlockSpec(bn.shape, lambda i: (0, 0, 0)),
        ],
        out_specs=pl.BlockSpec((BE, H), lambda i: (i, 0)),
        out_shape=jax.ShapeDtypeStruct((E, H), F32),
    )(x, w0, wr, b, bn)


def _edge_body(ea_ref, gi_ref, gj_ref, we_ref, be_ref,
               emw_ref, emb_ref, embn_ref,
               n1w_ref, n1wm_ref, n1b_ref, n1bm_ref, n1bn_ref,
               ean_ref, m_ref):
    ea = ea_ref[...]
    f_ij = _dot(ea, we_ref[...]) + be_ref[...]
    t = jnp.maximum(f_ij + gi_ref[...][:, :64] + gj_ref[...][:, :64], 0.0)
    ean = ea + _mlp_seq(t, _unstack(emw_ref, 5), _unstack(emb_ref, 5),
                        _unstack(embn_ref, 8))
    ean_ref[...] = ean
    mats = _unstack(n1w_ref, 4) + [n1wm_ref[...]]
    biases = _unstack(n1b_ref, 4) + [n1bm_ref[...]]
    m_ref[...] = _mlp_seq(ean, mats, biases, _unstack(n1bn_ref, 8))


def _edge_call(ea, gs, gd, we, be, emw, emb, embn, n1w, n1wm, n1b, n1bm, n1bn,
               E, H, BE):
    blk = pl.BlockSpec((BE, H), lambda i: (i, 0))
    blk2 = pl.BlockSpec((BE, 2 * H), lambda i: (i, 0))
    full = lambda a: pl.BlockSpec(a.shape, lambda i: (0,) * a.ndim)
    return pl.pallas_call(
        _edge_body,
        grid=(E // BE,),
        in_specs=[blk, blk2, blk2] + [full(a) for a in
                  (we, be, emw, emb, embn, n1w, n1wm, n1b, n1bm, n1bn)],
        out_specs=[blk, blk2],
        out_shape=[jax.ShapeDtypeStruct((E, H), F32),
                   jax.ShapeDtypeStruct((E, 2 * H), F32)],
    )(ea, gs, gd, we, be, emw, emb, embn, n1w, n1wm, n1b, n1bm, n1bn)


def _node_body(nf_ref, a0_ref, a1_ref, w_ref, b_ref, bn_ref, wio_ref,
               nfn_ref, ti_ref, tj_ref):
    n = nf_ref.shape[0]
    a = a0_ref[...][:n, :64] + a1_ref[...][:n, :64]
    nfn = nf_ref[...] + _mlp_seq(a, _unstack(w_ref, 5), _unstack(b_ref, 5),
                                 _unstack(bn_ref, 8))
    nfn_ref[...] = nfn
    tij = _dot(nfn, wio_ref[...])
    z = jnp.zeros_like(tij[:, :64])
    ti_ref[...] = jnp.concatenate([tij[:, :64], z], axis=1)
    tj_ref[...] = jnp.concatenate([tij[:, 64:], z], axis=1)


def _node_call(nf, agg0, agg1, w, b, bn, wio, N, H):
    t_sh = jax.ShapeDtypeStruct((N, 2 * H), F32)
    return pl.pallas_call(
        _node_body,
        out_shape=[jax.ShapeDtypeStruct((N, H), F32), t_sh, t_sh],
    )(nf, agg0, agg1, w, b, bn, wio)


def _sc_gather_pair(ti, tj, src, dst):
    N, W = ti.shape
    E = src.shape[0]
    CH = 80
    per_w = E // NW
    mesh = plsc.VectorSubcoreMesh(core_axis_name="c", subcore_axis_name="s")
    out = jax.ShapeDtypeStruct((E, W), F32)

    @functools.partial(
        pl.kernel,
        mesh=mesh,
        out_type=[out, out],
        scratch_types=[
            pltpu.VMEM((CH,), jnp.int32),
            pltpu.VMEM((CH,), jnp.int32),
            pltpu.VMEM((CH, W), F32),
            pltpu.VMEM((CH, W), F32),
            pltpu.SemaphoreType.DMA,
            pltpu.SemaphoreType.DMA,
        ],
    )
    def k(ti_hbm, tj_hbm, src_hbm, dst_hbm, gs_hbm, gd_hbm,
          idi_v, idj_v, ri_v, rj_v, semi, semj):
        wid = lax.axis_index("s") * NC + lax.axis_index("c")
        base0 = wid * per_w

        @pl.loop(0, per_w // CH)
        def _(kk):
            base = base0 + kk * CH
            pltpu.sync_copy(src_hbm.at[pl.ds(base, CH)], idi_v)
            pltpu.sync_copy(dst_hbm.at[pl.ds(base, CH)], idj_v)
            ci = pltpu.async_copy(ti_hbm.at[idi_v], ri_v, semi)
            cj = pltpu.async_copy(tj_hbm.at[idj_v], rj_v, semj)
            ci.wait()
            cj.wait()
            pltpu.sync_copy(ri_v, gs_hbm.at[pl.ds(base, CH)])
            pltpu.sync_copy(rj_v, gd_hbm.at[pl.ds(base, CH)])

    return k(ti, tj, src, dst)


def _sc_scatter(m, dst, zeros):
    E, W = m.shape
    N = zeros.shape[0]
    CH = 80
    rps = N // NS
    per_w = E // NW
    mesh = plsc.VectorSubcoreMesh(core_axis_name="c", subcore_axis_name="s")

    @functools.partial(
        pl.kernel,
        mesh=mesh,
        out_type=jax.ShapeDtypeStruct((NC, N, W), F32),
        scratch_types=[
            pltpu.VMEM((CH,), jnp.int32),
            pltpu.VMEM((CH, W), F32),
            pltpu.VMEM_SHARED((N, W), F32),
            pltpu.SemaphoreType.DMA,
        ],
    )
    def k(m_hbm, dst_hbm, z_hbm, out_hbm, idx_v, rows_v, acc_sh, sem):
        c = lax.axis_index("c")
        s = lax.axis_index("s")
        pltpu.sync_copy(z_hbm.at[pl.ds(s * rps, rps)],
                        acc_sh.at[pl.ds(s * rps, rps)])
        plsc.subcore_barrier()
        base0 = (c * NS + s) * per_w

        @pl.loop(0, per_w // CH)
        def _(kk):
            base = base0 + kk * CH
            pltpu.sync_copy(dst_hbm.at[pl.ds(base, CH)], idx_v)
            pltpu.sync_copy(m_hbm.at[pl.ds(base, CH)], rows_v)
            pltpu.sync_copy(rows_v, acc_sh.at[idx_v], add=True)

        plsc.subcore_barrier()
        pltpu.sync_copy(acc_sh.at[pl.ds(s * rps, rps)],
                        out_hbm.at[c].at[pl.ds(s * rps, rps)])

    return k(m, dst, zeros)


def kernel(node_feats, edge_index, edge_attr, params):
    node_feats = node_feats.astype(F32)
    edge_attr = edge_attr.astype(F32)
    src = edge_index[0].astype(jnp.int32)
    dst = edge_index[1].astype(jnp.int32)
    N, H = node_feats.shape[0], 64
    E = edge_attr.shape[0]
    BE = 4000

    stk = jnp.stack

    niW, nib, nibn = _mlp_pack(params["node_init"])
    eiW, eib, eibn = _mlp_pack(params["edge_init"])
    em = params["edge_model"]
    emW, emb, embn = _mlp_pack(em["mlp"])
    n1W, n1b, n1bn = _mlp_pack(params["node_model"]["mlp1"])
    n2W, n2b, n2bn = _mlp_pack(params["node_model"]["mlp2"])
    WeT = em["edge_lin"][0].T
    be = em["edge_lin"][1].reshape(1, H)
    wio = jnp.concatenate([em["node_in"].T, em["node_out"].T], axis=1)

    ni_args = (niW[0], stk(niW[1:]), stk(nib), stk(nibn))
    ei_args = (eiW[0], stk(eiW[1:]), stk(eib), stk(eibn))
    em_args = (stk(emW), stk(emb), stk(embn))
    n1_args = (stk(n1W[:4]), jnp.pad(n1W[4], ((0, 0), (0, H))),
               stk(n1b[:4]), jnp.pad(n1b[4], ((0, 0), (0, H))), stk(n1bn))
    n2_args = (stk(n2W), stk(n2b), stk(n2bn))

    nf, ti, tj = _node_init_call(node_feats, *ni_args, wio, N, H)
    ea = _edge_init_call(edge_attr, *ei_args, E, H, BE)
    zeros = jnp.zeros((NPAD, 2 * H), F32)

    for _ in range(DEPTH):
        gs, gd = _sc_gather_pair(ti, tj, src, dst)
        ea, msg = _edge_call(ea, gs, gd, WeT, be, *em_args, *n1_args, E, H, BE)
        agg2 = _sc_scatter(msg, dst, zeros)
        nf, ti, tj = _node_call(nf, agg2[0], agg2[1], *n2_args, wio, N, H)

    return nf, ea


# speedup vs baseline: 3.2040x; 1.3371x over previous
import functools

import jax
import jax.numpy as jnp
from jax import lax
from jax.experimental import pallas as pl
from jax.experimental.pallas import tpu as pltpu
from jax.experimental.pallas import tpu_sc as plsc

F32 = jnp.float32
DEPTH = 3
NC = 2
NS = 16
NW = NC * NS
NPAD = 10240


def _dot(x, w):
    return jnp.dot(x, w, preferred_element_type=F32)


def _mlp_pack(p, H=64):
    lins, bns = p["lins"], p["bns"]
    Ws = [W.T for (W, b) in lins]
    bs = [b.reshape(1, -1) for (W, b) in lins]
    bn = []
    for g, bb, m, v in bns:
        d = jnp.sqrt(v + 1e-5)
        bn += [m.reshape(1, H), d.reshape(1, H),
               g.reshape(1, H), bb.reshape(1, H)]
    return Ws, bs, bn


def _mlp_seq(x, mats, biases, bns):
    x = _dot(x, mats[0]) + biases[0]
    x = _dot(x, mats[1]) + biases[1]
    x = jnp.maximum((x - bns[0]) / bns[1] * bns[2] + bns[3], 0.0)
    x = _dot(x, mats[2]) + biases[2]
    x = _dot(x, mats[3]) + biases[3]
    x = jnp.maximum((x - bns[4]) / bns[5] * bns[6] + bns[7], 0.0)
    return _dot(x, mats[4]) + biases[4]


def _unstack(ref, n):
    return [ref[i] for i in range(n)]


def _node_init_body(x_ref, w0_ref, wr_ref, b_ref, bn_ref, wio_ref,
                    nf_ref, ti_ref, tj_ref):
    mats = [w0_ref[...]] + _unstack(wr_ref, 4)
    nf = _mlp_seq(x_ref[...], mats, _unstack(b_ref, 5), _unstack(bn_ref, 8))
    nf_ref[...] = nf
    tij = _dot(nf, wio_ref[...])
    z = jnp.zeros_like(tij[:, :64])
    ti_ref[...] = jnp.concatenate([tij[:, :64], z], axis=1)
    tj_ref[...] = jnp.concatenate([tij[:, 64:], z], axis=1)


def _node_init_call(x, w0, wr, b, bn, wio, N, H):
    t_sh = jax.ShapeDtypeStruct((N, 2 * H), F32)
    return pl.pallas_call(
        _node_init_body,
        out_shape=[jax.ShapeDtypeStruct((N, H), F32), t_sh, t_sh],
    )(x, w0, wr, b, bn, wio)


def _edge_init_body(x_ref, w0_ref, wr_ref, b_ref, bn_ref, ea_ref):
    mats = [w0_ref[...]] + _unstack(wr_ref, 4)
    ea_ref[...] = _mlp_seq(x_ref[...], mats, _unstack(b_ref, 5),
                           _unstack(bn_ref, 8))


def _edge_init_call(x, w0, wr, b, bn, E, H, BE):
    d_in = x.shape[1]
    return pl.pallas_call(
        _edge_init_body,
        grid=(E // BE,),
        in_specs=[
            pl.BlockSpec((BE, d_in), lambda i: (i, 0)),
            pl.BlockSpec(w0.shape, lambda i: (0, 0)),
            pl.BlockSpec(wr.shape, lambda i: (0, 0, 0)),
            pl.BlockSpec(b.shape, lambda i: (0, 0, 0)),
            pl.BlockSpec(bn.shape, lambda i: (0, 0, 0)),
        ],
        out_specs=pl.BlockSpec((BE, H), lambda i: (i, 0)),
        out_shape=jax.ShapeDtypeStruct((E, H), F32),
    )(x, w0, wr, b, bn)


def _edge_body(ea_ref, gi_ref, gj_ref, we_ref, be_ref,
               emw_ref, emb_ref, embn_ref,
               n1w_ref, n1wm_ref, n1b_ref, n1bm_ref, n1bn_ref,
               ean_ref, m_ref):
    ea = ea_ref[...]
    f_ij = _dot(ea, we_ref[...]) + be_ref[...]
    t = jnp.maximum(f_ij + gi_ref[...][:, :64] + gj_ref[...][:, :64], 0.0)
    ean = ea + _mlp_seq(t, _unstack(emw_ref, 5), _unstack(emb_ref, 5),
                        _unstack(embn_ref, 8))
    ean_ref[...] = ean
    mats = _unstack(n1w_ref, 4) + [n1wm_ref[...]]
    biases = _unstack(n1b_ref, 4) + [n1bm_ref[...]]
    m_ref[...] = _mlp_seq(ean, mats, biases, _unstack(n1bn_ref, 8))


def _edge_call(ea, gs, gd, we, be, emw, emb, embn, n1w, n1wm, n1b, n1bm, n1bn,
               E, H, BE):
    blk = pl.BlockSpec((BE, H), lambda i: (i, 0))
    blk2 = pl.BlockSpec((BE, 2 * H), lambda i: (i, 0))
    full = lambda a: pl.BlockSpec(a.shape, lambda i: (0,) * a.ndim)
    return pl.pallas_call(
        _edge_body,
        grid=(E // BE,),
        in_specs=[blk, blk2, blk2] + [full(a) for a in
                  (we, be, emw, emb, embn, n1w, n1wm, n1b, n1bm, n1bn)],
        out_specs=[blk, blk2],
        out_shape=[jax.ShapeDtypeStruct((E, H), F32),
                   jax.ShapeDtypeStruct((E, 2 * H), F32)],
    )(ea, gs, gd, we, be, emw, emb, embn, n1w, n1wm, n1b, n1bm, n1bn)


def _node_body(nf_ref, a0_ref, a1_ref, w_ref, b_ref, bn_ref, wio_ref,
               nfn_ref, ti_ref, tj_ref):
    n = nf_ref.shape[0]
    a = a0_ref[...][:n, :64] + a1_ref[...][:n, :64]
    nfn = nf_ref[...] + _mlp_seq(a, _unstack(w_ref, 5), _unstack(b_ref, 5),
                                 _unstack(bn_ref, 8))
    nfn_ref[...] = nfn
    tij = _dot(nfn, wio_ref[...])
    z = jnp.zeros_like(tij[:, :64])
    ti_ref[...] = jnp.concatenate([tij[:, :64], z], axis=1)
    tj_ref[...] = jnp.concatenate([tij[:, 64:], z], axis=1)


def _node_call(nf, agg0, agg1, w, b, bn, wio, N, H):
    t_sh = jax.ShapeDtypeStruct((N, 2 * H), F32)
    return pl.pallas_call(
        _node_body,
        out_shape=[jax.ShapeDtypeStruct((N, H), F32), t_sh, t_sh],
    )(nf, agg0, agg1, w, b, bn, wio)


def _sc_gather_pair(ti, tj, src, dst):
    N, W = ti.shape
    E = src.shape[0]
    CH = 80
    per_w = E // NW
    nch = per_w // CH
    mesh = plsc.VectorSubcoreMesh(core_axis_name="c", subcore_axis_name="s")
    out = jax.ShapeDtypeStruct((E, W), F32)

    @functools.partial(
        pl.kernel,
        mesh=mesh,
        out_type=[out, out],
        scratch_types=[
            pltpu.VMEM((per_w,), jnp.int32),
            pltpu.VMEM((per_w,), jnp.int32),
            pltpu.VMEM((CH, W), F32),
            pltpu.VMEM((CH, W), F32),
            pltpu.VMEM((CH, W), F32),
            pltpu.VMEM((CH, W), F32),
            pltpu.SemaphoreType.DMA,
            pltpu.SemaphoreType.DMA,
            pltpu.SemaphoreType.DMA,
            pltpu.SemaphoreType.DMA,
        ],
    )
    def k(ti_hbm, tj_hbm, src_hbm, dst_hbm, gs_hbm, gd_hbm,
          idi_v, idj_v, ri0, rj0, ri1, rj1, si0, sj0, si1, sj1):
        wid = lax.axis_index("s") * NC + lax.axis_index("c")
        base0 = wid * per_w
        pltpu.sync_copy(src_hbm.at[pl.ds(base0, per_w)], idi_v)
        pltpu.sync_copy(dst_hbm.at[pl.ds(base0, per_w)], idj_v)
        bufs = ((ri0, rj0, si0, sj0), (ri1, rj1, si1, sj1))

        def fire(kk, b):
            ri, rj, si, sj = bufs[b]
            s = pl.ds(kk * CH, CH)
            pltpu.async_copy(ti_hbm.at[idi_v.at[s]], ri, si)
            pltpu.async_copy(tj_hbm.at[idj_v.at[s]], rj, sj)

        def drain_store(kk, b):
            ri, rj, si, sj = bufs[b]
            dummy = ti_hbm.at[pl.ds(0, CH)]
            pltpu.make_async_copy(dummy, ri, si).wait()
            pltpu.make_async_copy(dummy, rj, sj).wait()
            s = pl.ds(base0 + kk * CH, CH)
            pltpu.sync_copy(ri, gs_hbm.at[s])
            pltpu.sync_copy(rj, gd_hbm.at[s])

        fire(0, 0)

        @pl.loop(0, (nch - 1) // 2)
        def _(t):
            k0 = 2 * t
            fire(k0 + 1, 1)
            drain_store(k0, 0)
            fire(k0 + 2, 0)
            drain_store(k0 + 1, 1)

        drain_store(nch - 1, 0)

    return k(ti, tj, src, dst)


def _sc_scatter(m, dst, zeros):
    E, W = m.shape
    N = zeros.shape[0]
    CH = 80
    rps = N // NS
    per_w = E // NW
    mesh = plsc.VectorSubcoreMesh(core_axis_name="c", subcore_axis_name="s")

    nch = per_w // CH

    @functools.partial(
        pl.kernel,
        mesh=mesh,
        out_type=jax.ShapeDtypeStruct((NC, N, W), F32),
        scratch_types=[
            pltpu.VMEM((CH,), jnp.int32),
            pltpu.VMEM((CH,), jnp.int32),
            pltpu.VMEM((CH, W), F32),
            pltpu.VMEM((CH, W), F32),
            pltpu.VMEM_SHARED((N, W), F32),
            pltpu.SemaphoreType.DMA,
            pltpu.SemaphoreType.DMA,
        ],
    )
    def k(m_hbm, dst_hbm, z_hbm, out_hbm, idx0, idx1, rows0, rows1,
          acc_sh, sm0, sm1):
        c = lax.axis_index("c")
        s = lax.axis_index("s")
        pltpu.sync_copy(z_hbm.at[pl.ds(s * rps, rps)],
                        acc_sh.at[pl.ds(s * rps, rps)])
        plsc.subcore_barrier()
        base0 = (c * NS + s) * per_w
        bufs = ((idx0, rows0, sm0), (idx1, rows1, sm1))

        def fire(kk, b):
            idx, rows, sm = bufs[b]
            base = base0 + kk * CH
            pltpu.sync_copy(dst_hbm.at[pl.ds(base, CH)], idx)
            pltpu.async_copy(m_hbm.at[pl.ds(base, CH)], rows, sm)

        def drain_scatter(kk, b):
            idx, rows, sm = bufs[b]
            pltpu.make_async_copy(m_hbm.at[pl.ds(0, CH)], rows, sm).wait()
            pltpu.sync_copy(rows, acc_sh.at[idx], add=True)

        fire(0, 0)

        @pl.loop(0, (nch - 1) // 2)
        def _(t):
            k0 = 2 * t
            fire(k0 + 1, 1)
            drain_scatter(k0, 0)
            fire(k0 + 2, 0)
            drain_scatter(k0 + 1, 1)

        drain_scatter(nch - 1, 0)

        plsc.subcore_barrier()
        pltpu.sync_copy(acc_sh.at[pl.ds(s * rps, rps)],
                        out_hbm.at[c].at[pl.ds(s * rps, rps)])

    return k(m, dst, zeros)


def kernel(node_feats, edge_index, edge_attr, params):
    node_feats = node_feats.astype(F32)
    edge_attr = edge_attr.astype(F32)
    src = edge_index[0].astype(jnp.int32)
    dst = edge_index[1].astype(jnp.int32)
    N, H = node_feats.shape[0], 64
    E = edge_attr.shape[0]
    BE = 4000

    stk = jnp.stack

    niW, nib, nibn = _mlp_pack(params["node_init"])
    eiW, eib, eibn = _mlp_pack(params["edge_init"])
    em = params["edge_model"]
    emW, emb, embn = _mlp_pack(em["mlp"])
    n1W, n1b, n1bn = _mlp_pack(params["node_model"]["mlp1"])
    n2W, n2b, n2bn = _mlp_pack(params["node_model"]["mlp2"])
    WeT = em["edge_lin"][0].T
    be = em["edge_lin"][1].reshape(1, H)
    wio = jnp.concatenate([em["node_in"].T, em["node_out"].T], axis=1)

    ni_args = (niW[0], stk(niW[1:]), stk(nib), stk(nibn))
    ei_args = (eiW[0], stk(eiW[1:]), stk(eib), stk(eibn))
    em_args = (stk(emW), stk(emb), stk(embn))
    n1_args = (stk(n1W[:4]), jnp.pad(n1W[4], ((0, 0), (0, H))),
               stk(n1b[:4]), jnp.pad(n1b[4], ((0, 0), (0, H))), stk(n1bn))
    n2_args = (stk(n2W), stk(n2b), stk(n2bn))

    nf, ti, tj = _node_init_call(node_feats, *ni_args, wio, N, H)
    ea = _edge_init_call(edge_attr, *ei_args, E, H, BE)
    zeros = jnp.zeros((NPAD, 2 * H), F32)

    for _ in range(DEPTH):
        gs, gd = _sc_gather_pair(ti, tj, src, dst)
        ea, msg = _edge_call(ea, gs, gd, WeT, be, *em_args, *n1_args, E, H, BE)
        agg2 = _sc_scatter(msg, dst, zeros)
        nf, ti, tj = _node_call(nf, agg2[0], agg2[1], *n2_args, wio, N, H)

    return nf, ea
